# baseline (speedup 1.0000x reference)
import jax
import jax.numpy as jnp
from jax import lax
from jax.experimental import pallas as pl
from jax.experimental.pallas import tpu as pltpu
from jax.experimental.pallas import tpu_sc as plsc

_EPS = 1e-06
_DT = 0.1
_T0 = 0.0
_NK = 512
_NP = 10000
_NF = 3
_NB = 64

_NC = 2
_NS = 16
_NW = _NC * _NS
_CW = 128
_NCH = _NP // _CW
_TAIL = _NP - _NCH * _CW
_NT = _NF * _NCH
_TMIN = _NT // _NW
_TXTRA = _NT - _TMIN * _NW
_PIECE = 64
_NPIECE = _CW // _PIECE


def _body(s_hbm, w_hbm, k_hbm, out_hbm, s_v, w_v, buf, out_buf, tail_v,
          sem_in, sem_out):
    cid = lax.axis_index("c")
    sid = lax.axis_index("s")
    wid = sid * _NC + cid

    pltpu.sync_copy(s_hbm, s_v)
    pltpu.sync_copy(w_hbm, w_v)

    lanes = lax.iota(jnp.int32, 16)
    rows_g = [g * 16 + lanes for g in range(_PIECE // 16)]

    t0 = wid * _TMIN + jnp.minimum(wid, _TXTRA)
    tend = t0 + _TMIN + jnp.where(wid < _TXTRA, 1, 0)

    def issue_in(t, p, slot):
        f = t // _NCH
        nn = (t - f * _NCH) * _CW
        pltpu.async_copy(
            k_hbm.at[f, pl.ds(nn + p * _PIECE, _PIECE), :], buf.at[slot],
            sem_in.at[slot])

    def wait_in(t, p, slot):
        f = t // _NCH
        nn = (t - f * _NCH) * _CW
        pltpu.make_async_copy(
            k_hbm.at[f, pl.ds(nn + p * _PIECE, _PIECE), :], buf.at[slot],
            sem_in.at[slot]).wait()

    def out_dma(t, slot):
        f = t // _NCH
        nn = (t - f * _NCH) * _CW
        return pltpu.make_async_copy(
            out_buf.at[slot], out_hbm.at[f, :, pl.ds(nn, _CW)],
            sem_out.at[slot])

    q0 = 2 * t0
    qend = 2 * tend
    issue_in(t0, 0, q0 % 3)
    q1 = jnp.minimum(q0 + 1, qend - 1)
    issue_in(q1 >> 1, q1 & 1, (q0 + 1) % 3)

    def do_piece(q, carry):
        t = q >> 1
        p = q & 1
        slot = q % 3
        tslot = t & 1
        qn = jnp.minimum(q + 2, qend - 1)
        issue_in(qn >> 1, qn & 1, (q + 2) % 3)
        wait_in(t, p, slot)

        @pl.when((p == 0) & (t >= t0 + 2))
        def _():
            out_dma(t - 2, tslot).wait()

        bufs = buf.at[slot]

        def do_b(i, c2):
            for j in range(4):
                b = i * 4 + j
                sb = s_v[b >> 3, pl.ds((b & 7) * 16, 16)]
                w1 = w_v[b >> 3, pl.ds((b & 7) * 16, 16)]
                w0 = 1.0 - w1
                sb1 = sb + 1
                for g in range(_PIECE // 16):
                    k0 = plsc.load_gather(bufs, [rows_g[g], sb])
                    k1 = plsc.load_gather(bufs, [rows_g[g], sb1])
                    out_buf[tslot, b, pl.ds(p * _PIECE + g * 16, 16)] = (
                        k0 * w0 + k1 * w1)
            return c2

        lax.fori_loop(0, _NB // 4, do_b, 0)

        @pl.when(p == 1)
        def _():
            out_dma(t, tslot).start()

        return carry

    lax.fori_loop(q0, qend, do_piece, 0)
    wait_in(tend - 1, 1, qend % 3)
    wait_in(tend - 1, 1, (qend + 1) % 3)
    for sl in range(2):
        out_dma(tend - 2 + sl, (tend - 2 + sl) & 1).wait()

    @pl.when(wid < _NF)
    def _():
        f = wid
        nn = _NCH * _CW
        pltpu.sync_copy(k_hbm.at[f, pl.ds(nn, _TAIL), :], buf.at[0, pl.ds(0, _TAIL), :])

        def tail_b(b, c2):
            sb = s_v[b >> 3, pl.ds((b & 7) * 16, 16)]
            w1 = w_v[b >> 3, pl.ds((b & 7) * 16, 16)]
            w0 = 1.0 - w1
            k0 = plsc.load_gather(buf.at[0], [lanes, sb])
            k1 = plsc.load_gather(buf.at[0], [lanes, sb + 1])
            tail_v[b, :] = k0 * w0 + k1 * w1
            return c2

        lax.fori_loop(0, _NB, tail_b, 0)
        pltpu.sync_copy(tail_v, out_hbm.at[f, :, pl.ds(nn, _TAIL)])


def _sc_spline(s16, w16, kt):
    mesh = plsc.VectorSubcoreMesh(core_axis_name="c", subcore_axis_name="s",
                                  num_cores=_NC, num_subcores=_NS)
    f = pl.kernel(
        _body,
        out_type=jax.ShapeDtypeStruct((_NF, _NB, _NP), jnp.float32),
        mesh=mesh,
        scratch_types=[
            pltpu.VMEM((8, 128), jnp.int32),
            pltpu.VMEM((8, 128), jnp.float32),
            pltpu.VMEM((3, _PIECE, _NK), jnp.float32),
            pltpu.VMEM((2, _NB, _CW), jnp.float32),
            pltpu.VMEM((_NB, _TAIL), jnp.float32),
            pltpu.SemaphoreType.DMA((3,)),
            pltpu.SemaphoreType.DMA((2,)),
        ],
        compiler_params=pltpu.CompilerParams(use_tc_tiling_on_sc=True,
                                             needs_layout_passes=False),
    )
    return f(s16, w16, kt)


def kernel(timestamps, knots):
    t_hi = _T0 + _DT * (_NK - 1)
    ts = jnp.clip(timestamps, _T0 + _EPS, t_hi - _EPS)
    nt = (ts - _T0) / _DT
    s = jnp.floor(nt).astype(jnp.int32)
    u = (nt - s.astype(jnp.float32))
    s16 = jnp.broadcast_to(s[:, None], (_NB, 16)).reshape(8, 128)
    w16 = jnp.broadcast_to(u[:, None], (_NB, 16)).reshape(8, 128)
    kt = jnp.transpose(knots, (2, 0, 1))
    out = _sc_spline(s16, w16, kt)
    return jnp.transpose(out, (1, 2, 0))


# speedup vs baseline: 1.3251x; 1.3251x over previous
import jax
import jax.numpy as jnp
from jax import lax
from jax.experimental import pallas as pl
from jax.experimental.pallas import tpu as pltpu
from jax.experimental.pallas import tpu_sc as plsc

_EPS = 1e-06
_DT = 0.1
_T0 = 0.0
_NK = 512
_NP = 10000
_NF = 3
_NB = 64

_NC = 2
_NS = 16
_NW = _NC * _NS
_CW = 128
_NCH = _NP // _CW
_TAIL = _NP - _NCH * _CW
_NT = _NF * _NCH
_TMIN = _NT // _NW
_TXTRA = _NT - _TMIN * _NW
_PIECE = 64
_NPIECE = _CW // _PIECE


def _body(s_hbm, w_hbm, k_hbm, out_hbm, s_v, w_v, buf, out_buf, tail_v,
          sem_in, sem_out):
    cid = lax.axis_index("c")
    sid = lax.axis_index("s")
    wid = sid * _NC + cid

    pltpu.sync_copy(s_hbm, s_v)
    pltpu.sync_copy(w_hbm, w_v)

    lanes = lax.iota(jnp.int32, 16)
    rows_g = [g * 16 + lanes for g in range(_PIECE // 16)]

    t0 = wid * _TMIN + jnp.minimum(wid, _TXTRA)
    tend = t0 + _TMIN + jnp.where(wid < _TXTRA, 1, 0)

    def issue_in(t, p, slot):
        f = t // _NCH
        nn = (t - f * _NCH) * _CW
        pltpu.async_copy(
            k_hbm.at[f, pl.ds(nn + p * _PIECE, _PIECE), :], buf.at[slot],
            sem_in.at[slot])

    def wait_in(t, p, slot):
        f = t // _NCH
        nn = (t - f * _NCH) * _CW
        pltpu.make_async_copy(
            k_hbm.at[f, pl.ds(nn + p * _PIECE, _PIECE), :], buf.at[slot],
            sem_in.at[slot]).wait()

    def out_dma(t, slot):
        f = t // _NCH
        nn = (t - f * _NCH) * _CW
        return pltpu.make_async_copy(
            out_buf.at[slot], out_hbm.at[f, :, pl.ds(nn, _CW)],
            sem_out.at[slot])

    qend = 2 * tend
    issue_in(t0, 0, 0)

    def do_piece(q, carry):
        t = q >> 1
        p = q & 1
        slot = q & 1
        tslot = t & 1
        qn = jnp.minimum(q + 1, qend - 1)
        issue_in(qn >> 1, qn & 1, 1 - slot)
        wait_in(t, p, slot)

        @pl.when((p == 0) & (t >= t0 + 2))
        def _():
            out_dma(t - 2, tslot).wait()

        bufs = buf.at[slot]

        @plsc.parallel_loop(0, _NB, step=1, unroll=4)
        def _(b):
            sb = s_v[b >> 3, pl.ds((b & 7) * 16, 16)]
            w1 = w_v[b >> 3, pl.ds((b & 7) * 16, 16)]
            w0 = 1.0 - w1
            sb1 = sb + 1
            for g in range(_PIECE // 16):
                k0 = plsc.load_gather(bufs, [rows_g[g], sb])
                k1 = plsc.load_gather(bufs, [rows_g[g], sb1])
                out_buf[tslot, b, pl.ds(p * _PIECE + g * 16, 16)] = (
                    k0 * w0 + k1 * w1)

        @pl.when(p == 1)
        def _():
            out_dma(t, tslot).start()

        return carry

    lax.fori_loop(2 * t0, qend, do_piece, 0)
    wait_in(tend - 1, 1, 0)
    for sl in range(2):
        out_dma(tend - 2 + sl, (tend - 2 + sl) & 1).wait()

    @pl.when(wid < _NF)
    def _():
        f = wid
        nn = _NCH * _CW
        pltpu.sync_copy(k_hbm.at[f, pl.ds(nn, _TAIL), :], buf.at[0, pl.ds(0, _TAIL), :])

        def tail_b(b, c2):
            sb = s_v[b >> 3, pl.ds((b & 7) * 16, 16)]
            w1 = w_v[b >> 3, pl.ds((b & 7) * 16, 16)]
            w0 = 1.0 - w1
            k0 = plsc.load_gather(buf.at[0], [lanes, sb])
            k1 = plsc.load_gather(buf.at[0], [lanes, sb + 1])
            tail_v[b, :] = k0 * w0 + k1 * w1
            return c2

        lax.fori_loop(0, _NB, tail_b, 0)
        pltpu.sync_copy(tail_v, out_hbm.at[f, :, pl.ds(nn, _TAIL)])


def _sc_spline(s16, w16, kt):
    mesh = plsc.VectorSubcoreMesh(core_axis_name="c", subcore_axis_name="s",
                                  num_cores=_NC, num_subcores=_NS)
    f = pl.kernel(
        _body,
        out_type=jax.ShapeDtypeStruct((_NF, _NB, _NP), jnp.float32),
        mesh=mesh,
        scratch_types=[
            pltpu.VMEM((8, 128), jnp.int32),
            pltpu.VMEM((8, 128), jnp.float32),
            pltpu.VMEM((2, _PIECE, _NK), jnp.float32),
            pltpu.VMEM((2, _NB, _CW), jnp.float32),
            pltpu.VMEM((_NB, _TAIL), jnp.float32),
            pltpu.SemaphoreType.DMA((2,)),
            pltpu.SemaphoreType.DMA((2,)),
        ],
        compiler_params=pltpu.CompilerParams(use_tc_tiling_on_sc=True,
                                             needs_layout_passes=False),
    )
    return f(s16, w16, kt)


def kernel(timestamps, knots):
    t_hi = _T0 + _DT * (_NK - 1)
    ts = jnp.clip(timestamps, _T0 + _EPS, t_hi - _EPS)
    nt = (ts - _T0) / _DT
    s = jnp.floor(nt).astype(jnp.int32)
    u = (nt - s.astype(jnp.float32))
    s16 = jnp.broadcast_to(s[:, None], (_NB, 16)).reshape(8, 128)
    w16 = jnp.broadcast_to(u[:, None], (_NB, 16)).reshape(8, 128)
    kt = jnp.transpose(knots, (2, 0, 1))
    out = _sc_spline(s16, w16, kt)
    return jnp.transpose(out, (1, 2, 0))


# speedup vs baseline: 1.3800x; 1.0414x over previous
import jax
import jax.numpy as jnp
from jax import lax
from jax.experimental import pallas as pl
from jax.experimental.pallas import tpu as pltpu
from jax.experimental.pallas import tpu_sc as plsc

_EPS = 1e-06
_DT = 0.1
_T0 = 0.0
_NK = 512
_NP = 10000
_NF = 3
_NB = 64

_NC = 2
_NS = 16
_NW = _NC * _NS
_CW = 128
_NCH = _NP // _CW
_TAIL = _NP - _NCH * _CW
_NT = _NF * _NCH
_TMIN = _NT // _NW
_TXTRA = _NT - _TMIN * _NW
_PIECE = 64
_NPIECE = _CW // _PIECE


def _body(s_hbm, w_hbm, k_hbm, out_hbm, s_v, w_v, buf, out_buf, tail_v,
          sem_in, sem_out):
    cid = lax.axis_index("c")
    sid = lax.axis_index("s")
    wid = sid * _NC + cid

    pltpu.sync_copy(s_hbm, s_v)
    pltpu.sync_copy(w_hbm, w_v)

    lanes = lax.iota(jnp.int32, 16)
    rows_g = [g * 16 + lanes for g in range(_PIECE // 16)]

    t0 = wid * _TMIN + jnp.minimum(wid, _TXTRA)
    tend = t0 + _TMIN + jnp.where(wid < _TXTRA, 1, 0)

    def issue_in(t, p, slot):
        f = t // _NCH
        nn = (t - f * _NCH) * _CW
        pltpu.async_copy(
            k_hbm.at[f, pl.ds(nn + p * _PIECE, _PIECE), :], buf.at[slot],
            sem_in.at[slot])

    def wait_in(t, p, slot):
        f = t // _NCH
        nn = (t - f * _NCH) * _CW
        pltpu.make_async_copy(
            k_hbm.at[f, pl.ds(nn + p * _PIECE, _PIECE), :], buf.at[slot],
            sem_in.at[slot]).wait()

    def out_dma(t, slot):
        f = t // _NCH
        nn = (t - f * _NCH) * _CW
        return pltpu.make_async_copy(
            out_buf.at[slot], out_hbm.at[f, :, pl.ds(nn, _CW)],
            sem_out.at[slot])

    qend = 2 * tend
    issue_in(t0, 0, 0)

    def do_piece(q, carry):
        t = q >> 1
        p = q & 1
        slot = q & 1
        tslot = t & 1
        qn = jnp.minimum(q + 1, qend - 1)
        issue_in(qn >> 1, qn & 1, 1 - slot)
        wait_in(t, p, slot)

        @pl.when((p == 0) & (t >= t0 + 2))
        def _():
            out_dma(t - 2, tslot).wait()

        bufs = buf.at[slot]

        @plsc.parallel_loop(0, _NB, step=1, unroll=8)
        def _(b):
            sb = s_v[b >> 3, pl.ds((b & 7) * 16, 16)]
            w1 = w_v[b >> 3, pl.ds((b & 7) * 16, 16)]
            w0 = 1.0 - w1
            sb1 = sb + 1
            for g in range(_PIECE // 16):
                k0 = plsc.load_gather(bufs, [rows_g[g], sb])
                k1 = plsc.load_gather(bufs, [rows_g[g], sb1])
                out_buf[tslot, b, pl.ds(p * _PIECE + g * 16, 16)] = (
                    k0 * w0 + k1 * w1)

        @pl.when(p == 1)
        def _():
            out_dma(t, tslot).start()

        return carry

    lax.fori_loop(2 * t0, qend, do_piece, 0)
    wait_in(tend - 1, 1, 0)
    for sl in range(2):
        out_dma(tend - 2 + sl, (tend - 2 + sl) & 1).wait()

    @pl.when(wid < _NF)
    def _():
        f = wid
        nn = _NCH * _CW
        pltpu.sync_copy(k_hbm.at[f, pl.ds(nn, _TAIL), :], buf.at[0, pl.ds(0, _TAIL), :])

        def tail_b(b, c2):
            sb = s_v[b >> 3, pl.ds((b & 7) * 16, 16)]
            w1 = w_v[b >> 3, pl.ds((b & 7) * 16, 16)]
            w0 = 1.0 - w1
            k0 = plsc.load_gather(buf.at[0], [lanes, sb])
            k1 = plsc.load_gather(buf.at[0], [lanes, sb + 1])
            tail_v[b, :] = k0 * w0 + k1 * w1
            return c2

        lax.fori_loop(0, _NB, tail_b, 0)
        pltpu.sync_copy(tail_v, out_hbm.at[f, :, pl.ds(nn, _TAIL)])


def _sc_spline(s16, w16, kt):
    mesh = plsc.VectorSubcoreMesh(core_axis_name="c", subcore_axis_name="s",
                                  num_cores=_NC, num_subcores=_NS)
    f = pl.kernel(
        _body,
        out_type=jax.ShapeDtypeStruct((_NF, _NB, _NP), jnp.float32),
        mesh=mesh,
        scratch_types=[
            pltpu.VMEM((8, 128), jnp.int32),
            pltpu.VMEM((8, 128), jnp.float32),
            pltpu.VMEM((2, _PIECE, _NK), jnp.float32),
            pltpu.VMEM((2, _NB, _CW), jnp.float32),
            pltpu.VMEM((_NB, _TAIL), jnp.float32),
            pltpu.SemaphoreType.DMA((2,)),
            pltpu.SemaphoreType.DMA((2,)),
        ],
        compiler_params=pltpu.CompilerParams(use_tc_tiling_on_sc=True,
                                             needs_layout_passes=False),
    )
    return f(s16, w16, kt)


def kernel(timestamps, knots):
    t_hi = _T0 + _DT * (_NK - 1)
    ts = jnp.clip(timestamps, _T0 + _EPS, t_hi - _EPS)
    nt = (ts - _T0) / _DT
    s = jnp.floor(nt).astype(jnp.int32)
    u = (nt - s.astype(jnp.float32))
    s16 = jnp.broadcast_to(s[:, None], (_NB, 16)).reshape(8, 128)
    w16 = jnp.broadcast_to(u[:, None], (_NB, 16)).reshape(8, 128)
    kt = jnp.transpose(knots, (2, 0, 1))
    out = _sc_spline(s16, w16, kt)
    return jnp.transpose(out, (1, 2, 0))
